# Initial kernel scaffold; baseline (speedup 1.0000x reference)
import jax
import jax.numpy as jnp
from jax.experimental import pallas as pl


def kernel(atom_features, bond_features, molecule_features, Wi_w, Wi_b, Wm_w, Wm_b, Wa_w, Wa_b, bond_index, atom_to_molecule, b2rev):
    raise NotImplementedError("write your pallas kernel here")


# speedup vs baseline: 1.3384x; 1.3384x over previous
import functools

import jax
import jax.numpy as jnp
from jax import lax
from jax.experimental import pallas as pl
from jax.experimental.pallas import tpu as pltpu
from jax.experimental.pallas import tpu_sc as plsc

N_ATOMS = 10000
N_MOL = 500
H = 128
BOND_DIM = 16
DEPTH = 3

HALF = 80000
HALF_PAD = 81920
E_PAD = 2 * HALF_PAD
PAD = HALF_PAD - HALF

NC, NS = 2, 16
NW = NC * NS
PER_W = E_PAD // NW
CHUNK = 128
N_CHUNKS = PER_W // CHUNK
SUP = 512
SPC = SUP // CHUNK
N_SUP = PER_W // SUP
SUP_S = 256
SPC_S = SUP_S // CHUNK
N_SUP_S = PER_W // SUP_S

N_A_PAD = 10240
ROWS_PER_SUB = N_A_PAD // NS
ZCHUNK = 64

BLK = 1024
NBLK = E_PAD // BLK
BLKA = 1000
NBLKA = N_ATOMS // BLKA

_HIGHEST = lax.Precision.HIGHEST


def _worker_id():
    return lax.axis_index("c") * NS + lax.axis_index("s")


_sc_cache = {}


def _sc_kernels():
    if _sc_cache:
        return _sc_cache["g1"], _sc_cache["g2"], _sc_cache["scat"]

    mesh = plsc.VectorSubcoreMesh(core_axis_name="c", subcore_axis_name="s",
                                  num_cores=NC, num_subcores=NS)

    @functools.partial(
        pl.kernel,
        out_type=jax.ShapeDtypeStruct((E_PAD, H), jnp.float32),
        mesh=mesh,
        scratch_types=[
            pltpu.VMEM((N_CHUNKS, CHUNK), jnp.int32),
            pltpu.VMEM((SUP, H), jnp.float32),
            pltpu.SemaphoreType.DMA,
        ],
    )
    def _sc_gather1(tab_hbm, idx_hbm, out_hbm, idx_v, rows_v, sem):
        wid = _worker_id()
        base = wid * PER_W
        pltpu.sync_copy(idx_hbm.at[wid], idx_v)

        def sup(j, carry):
            descs = [
                pltpu.async_copy(
                    tab_hbm.at[idx_v.at[j * SPC + k]],
                    rows_v.at[pl.ds(k * CHUNK, CHUNK)],
                    sem,
                )
                for k in range(SPC)
            ]
            for d in descs:
                d.wait()
            pltpu.sync_copy(rows_v, out_hbm.at[pl.ds(base + j * SUP, SUP)])
            return carry

        lax.fori_loop(0, N_SUP, sup, None)

    @functools.partial(
        pl.kernel,
        out_type=(
            jax.ShapeDtypeStruct((E_PAD, H), jnp.float32),
            jax.ShapeDtypeStruct((E_PAD, H), jnp.float32),
        ),
        mesh=mesh,
        scratch_types=[
            pltpu.VMEM((N_CHUNKS, CHUNK), jnp.int32),
            pltpu.VMEM((N_CHUNKS, CHUNK), jnp.int32),
            pltpu.VMEM((SUP, H), jnp.float32),
            pltpu.SemaphoreType.DMA,
        ],
    )
    def _sc_gather2(tab_hbm, idxa_hbm, idxb_hbm, ga_hbm, gb_hbm,
                    idxa_v, idxb_v, rows_v, sem):
        wid = _worker_id()
        base = wid * PER_W
        pltpu.sync_copy(idxa_hbm.at[wid], idxa_v)
        pltpu.sync_copy(idxb_hbm.at[wid], idxb_v)

        def sup(j, carry):
            for idx_v, out_hbm in ((idxa_v, ga_hbm), (idxb_v, gb_hbm)):
                descs = [
                    pltpu.async_copy(
                        tab_hbm.at[idx_v.at[j * SPC + k]],
                        rows_v.at[pl.ds(k * CHUNK, CHUNK)],
                        sem,
                    )
                    for k in range(SPC)
                ]
                for d in descs:
                    d.wait()
                pltpu.sync_copy(rows_v, out_hbm.at[pl.ds(base + j * SUP, SUP)])
            return carry

        lax.fori_loop(0, N_SUP, sup, None)

    @functools.partial(
        pl.kernel,
        out_type=jax.ShapeDtypeStruct((2 * N_A_PAD, H), jnp.float32),
        mesh=mesh,
        scratch_types=[
            pltpu.VMEM((N_CHUNKS, CHUNK), jnp.int32),
            pltpu.VMEM((SUP_S, H), jnp.float32),
            pltpu.VMEM((ZCHUNK, H), jnp.float32),
            pltpu.VMEM_SHARED((N_A_PAD, H), jnp.float32),
            pltpu.SemaphoreType.DMA,
        ],
    )
    def _sc_scatter(rows_hbm, idx_hbm, out_hbm, idx_v, rows_v, zbuf_v,
                    agg_sh, sem):
        c = lax.axis_index("c")
        s = lax.axis_index("s")
        wid = c * NS + s
        base = wid * PER_W

        def zrow(i, carry):
            for cc in range(H // 16):
                zbuf_v[i, pl.ds(cc * 16, 16)] = jnp.zeros((16,), jnp.float32)
            return carry

        lax.fori_loop(0, ZCHUNK, zrow, None)

        def zcp(k, carry):
            r = s * ROWS_PER_SUB + k * ZCHUNK
            pltpu.sync_copy(zbuf_v, agg_sh.at[pl.ds(r, ZCHUNK)])
            return carry

        lax.fori_loop(0, ROWS_PER_SUB // ZCHUNK, zcp, None)
        plsc.subcore_barrier()

        pltpu.sync_copy(idx_hbm.at[wid], idx_v)

        def sup(j, carry):
            pltpu.sync_copy(rows_hbm.at[pl.ds(base + j * SUP_S, SUP_S)], rows_v)
            descs = [
                pltpu.async_copy(
                    rows_v.at[pl.ds(k * CHUNK, CHUNK)],
                    agg_sh.at[idx_v.at[j * SPC_S + k]],
                    sem,
                    add=True,
                )
                for k in range(SPC_S)
            ]
            for d in descs:
                d.wait()
            return carry

        lax.fori_loop(0, N_SUP_S, sup, None)
        plsc.subcore_barrier()

        def cp(k, carry):
            r = s * ROWS_PER_SUB + k * ZCHUNK
            pltpu.sync_copy(agg_sh.at[pl.ds(r, ZCHUNK)], zbuf_v)
            pltpu.sync_copy(zbuf_v, out_hbm.at[pl.ds(c * N_A_PAD + r, ZCHUNK)])
            return carry

        lax.fori_loop(0, ROWS_PER_SUB // ZCHUNK, cp, None)

    _sc_cache.update(g1=_sc_gather1, g2=_sc_gather2, scat=_sc_scatter)
    return _sc_cache["g1"], _sc_cache["g2"], _sc_cache["scat"]


def _pad_mask(i, blk):
    row = i * blk + lax.broadcasted_iota(jnp.int32, (blk, H), 0)
    return (row % HALF_PAD) < HALF


def _init_body(bond_ref, xg_ref, wb_ref, wx_ref, b_ref, out_ref):
    acc = jnp.dot(bond_ref[...], wb_ref[...],
                  preferred_element_type=jnp.float32, precision=_HIGHEST)
    acc += jnp.dot(xg_ref[...], wx_ref[...],
                   preferred_element_type=jnp.float32, precision=_HIGHEST)
    h0 = jnp.maximum(acc + b_ref[...], 0.0)
    out_ref[...] = jnp.where(_pad_mask(pl.program_id(0), BLK), h0, 0.0)


_tc_init = pl.pallas_call(
    _init_body,
    grid=(NBLK,),
    in_specs=[
        pl.BlockSpec((BLK, BOND_DIM), lambda i: (i, 0)),
        pl.BlockSpec((BLK, H), lambda i: (i, 0)),
        pl.BlockSpec((BOND_DIM, H), lambda i: (0, 0)),
        pl.BlockSpec((H, H), lambda i: (0, 0)),
        pl.BlockSpec((1, H), lambda i: (0, 0)),
    ],
    out_specs=pl.BlockSpec((BLK, H), lambda i: (i, 0)),
    out_shape=jax.ShapeDtypeStruct((E_PAD, H), jnp.float32),
)


def _depth_body(h0_ref, ga_ref, gb_ref, hrev_ref, w_ref, b_ref, out_ref):
    m = ga_ref[...] + gb_ref[...] - hrev_ref[...]
    acc = jnp.dot(m, w_ref[...],
                  preferred_element_type=jnp.float32, precision=_HIGHEST)
    h = jnp.maximum(h0_ref[...] + acc + b_ref[...], 0.0)
    out_ref[...] = jnp.where(_pad_mask(pl.program_id(0), BLK), h, 0.0)


_tc_depth = pl.pallas_call(
    _depth_body,
    grid=(NBLK,),
    in_specs=[
        pl.BlockSpec((BLK, H), lambda i: (i, 0)),
        pl.BlockSpec((BLK, H), lambda i: (i, 0)),
        pl.BlockSpec((BLK, H), lambda i: (i, 0)),
        pl.BlockSpec((BLK, H), lambda i: ((i + NBLK // 2) % NBLK, 0)),
        pl.BlockSpec((H, H), lambda i: (0, 0)),
        pl.BlockSpec((1, H), lambda i: (0, 0)),
    ],
    out_specs=pl.BlockSpec((BLK, H), lambda i: (i, 0)),
    out_shape=jax.ShapeDtypeStruct((E_PAD, H), jnp.float32),
)


def _final_body(atom_ref, agga_ref, aggb_ref, a2m_ref, molf_ref,
                wx_ref, wm_ref, b_ref, out_ref):
    i = pl.program_id(0)
    mv = agga_ref[...] + aggb_ref[...]
    hv = jnp.dot(atom_ref[...], wx_ref[...],
                 preferred_element_type=jnp.float32, precision=_HIGHEST)
    hv += jnp.dot(mv, wm_ref[...],
                  preferred_element_type=jnp.float32, precision=_HIGHEST)
    hv = jnp.maximum(hv + b_ref[...], 0.0)
    seg = a2m_ref[0, 0, :]
    onehot = (lax.broadcasted_iota(jnp.int32, (N_MOL, BLKA), 0)
              == seg[None, :]).astype(jnp.float32)
    contrib = jnp.dot(onehot, hv,
                      preferred_element_type=jnp.float32, precision=_HIGHEST)

    @pl.when(i == 0)
    def _():
        out_ref[:, :H] = contrib
        out_ref[:, H:] = molf_ref[...]

    @pl.when(i != 0)
    def _():
        out_ref[:, :H] = out_ref[:, :H] + contrib


_tc_final = pl.pallas_call(
    _final_body,
    grid=(NBLKA,),
    in_specs=[
        pl.BlockSpec((BLKA, H), lambda i: (i, 0)),
        pl.BlockSpec((BLKA, H), lambda i: (i, 0)),
        pl.BlockSpec((BLKA, H), lambda i: (i, 0)),
        pl.BlockSpec((1, 1, BLKA), lambda i: (i, 0, 0)),
        pl.BlockSpec((N_MOL, H), lambda i: (0, 0)),
        pl.BlockSpec((H, H), lambda i: (0, 0)),
        pl.BlockSpec((H, H), lambda i: (0, 0)),
        pl.BlockSpec((1, H), lambda i: (0, 0)),
    ],
    out_specs=pl.BlockSpec((N_MOL, 2 * H), lambda i: (0, 0)),
    out_shape=jax.ShapeDtypeStruct((N_MOL, 2 * H), jnp.float32),
)


def _mid_pad(x, fill):
    z = jnp.full((PAD,) + x.shape[1:], fill, dtype=x.dtype)
    return jnp.concatenate([x[:HALF], z, x[HALF:], z], axis=0)


def kernel(atom_features, bond_features, molecule_features, Wi_w, Wi_b,
           Wm_w, Wm_b, Wa_w, Wa_b, bond_index, atom_to_molecule, b2rev):
    src = bond_index[0].astype(jnp.int32)
    dst = bond_index[1].astype(jnp.int32)

    src_p = _mid_pad(src, 0)
    dst_p = _mid_pad(dst, 0)
    bond_p = _mid_pad(bond_features, 0.0)

    src_t = src_p.reshape(NW, N_CHUNKS, CHUNK)
    srcn_t = (src_p + N_A_PAD).reshape(NW, N_CHUNKS, CHUNK)
    dst_t = dst_p.reshape(NW, N_CHUNKS, CHUNK)

    wi_t = Wi_w.T
    wb_t = wi_t[:BOND_DIM]
    wx_t = wi_t[BOND_DIM:]
    wm_t = Wm_w.T
    wa_t = Wa_w.T
    wax_t = wa_t[:H]
    wam_t = wa_t[H:]

    wi_b = Wi_b.reshape(1, H)
    wm_b = Wm_b.reshape(1, H)
    wa_b = Wa_b.reshape(1, H)

    a2m_t = atom_to_molecule.astype(jnp.int32).reshape(NBLKA, 1, BLKA)

    sc_gather1, sc_gather2, sc_scatter = _sc_kernels()

    xg = sc_gather1(atom_features, src_t)
    h0 = _tc_init(bond_p, xg, wb_t, wx_t, wi_b)

    h = h0
    for _ in range(DEPTH):
        agg2 = sc_scatter(h, dst_t)
        ga, gb = sc_gather2(agg2, src_t, srcn_t)
        h = _tc_depth(h0, ga, gb, h, wm_t, wm_b)

    agg2f = sc_scatter(h, src_t)
    return _tc_final(atom_features, agg2f[:N_ATOMS], agg2f[N_A_PAD:N_A_PAD + N_ATOMS],
                     a2m_t, molecule_features, wax_t, wam_t, wa_b)


# speedup vs baseline: 1.6747x; 1.2513x over previous
import functools

import jax
import jax.numpy as jnp
from jax import lax
from jax.experimental import pallas as pl
from jax.experimental.pallas import tpu as pltpu
from jax.experimental.pallas import tpu_sc as plsc

N_ATOMS = 10000
N_MOL = 500
H = 128
BOND_DIM = 16
DEPTH = 3

HALF = 80000
HALF_PAD = 81920
E_PAD = 2 * HALF_PAD
PAD = HALF_PAD - HALF

NC, NS = 2, 16
NW = NC * NS
PER_W = E_PAD // NW
CHUNK = 128
N_CHUNKS = PER_W // CHUNK
SUP = 512
SPC = SUP // CHUNK
N_SUP = PER_W // SUP
SUP_S = 256
SPC_S = SUP_S // CHUNK
N_SUP_S = PER_W // SUP_S

N_A_PAD = 10240
ROWS_PER_SUB = N_A_PAD // NS
ZCHUNK = 64

BLK = 1024
NBLK = E_PAD // BLK
BLKA = 1000
NBLKA = N_ATOMS // BLKA

_HIGHEST = lax.Precision.HIGHEST


def _worker_id():
    return lax.axis_index("c") * NS + lax.axis_index("s")


_sc_cache = {}


def _sc_kernels():
    if _sc_cache:
        return _sc_cache["g1"], _sc_cache["scat"]

    mesh = plsc.VectorSubcoreMesh(core_axis_name="c", subcore_axis_name="s",
                                  num_cores=NC, num_subcores=NS)

    @functools.partial(
        pl.kernel,
        out_type=jax.ShapeDtypeStruct((E_PAD, H), jnp.float32),
        mesh=mesh,
        scratch_types=[
            pltpu.VMEM((N_CHUNKS, CHUNK), jnp.int32),
            pltpu.VMEM((SUP, H), jnp.float32),
            pltpu.SemaphoreType.DMA,
        ],
    )
    def _sc_gather1(tab_hbm, idx_hbm, out_hbm, idx_v, rows_v, sem):
        wid = _worker_id()
        base = wid * PER_W
        pltpu.sync_copy(idx_hbm.at[wid], idx_v)

        def sup(j, carry):
            descs = [
                pltpu.async_copy(
                    tab_hbm.at[idx_v.at[j * SPC + k]],
                    rows_v.at[pl.ds(k * CHUNK, CHUNK)],
                    sem,
                )
                for k in range(SPC)
            ]
            for d in descs:
                d.wait()
            pltpu.sync_copy(rows_v, out_hbm.at[pl.ds(base + j * SUP, SUP)])
            return carry

        lax.fori_loop(0, N_SUP, sup, None)

    @functools.partial(
        pl.kernel,
        out_type=jax.ShapeDtypeStruct((2 * N_A_PAD, H), jnp.float32),
        mesh=mesh,
        scratch_types=[
            pltpu.VMEM((N_CHUNKS, CHUNK), jnp.int32),
            pltpu.VMEM((SUP_S, H), jnp.float32),
            pltpu.VMEM((ZCHUNK, H), jnp.float32),
            pltpu.VMEM_SHARED((N_A_PAD, H), jnp.float32),
            pltpu.SemaphoreType.DMA,
        ],
    )
    def _sc_scatter(rows_hbm, idx_hbm, out_hbm, idx_v, rows_v, zbuf_v,
                    agg_sh, sem):
        c = lax.axis_index("c")
        s = lax.axis_index("s")
        wid = c * NS + s
        base = wid * PER_W

        def zrow(i, carry):
            for cc in range(H // 16):
                zbuf_v[i, pl.ds(cc * 16, 16)] = jnp.zeros((16,), jnp.float32)
            return carry

        lax.fori_loop(0, ZCHUNK, zrow, None)

        def zcp(k, carry):
            r = s * ROWS_PER_SUB + k * ZCHUNK
            pltpu.sync_copy(zbuf_v, agg_sh.at[pl.ds(r, ZCHUNK)])
            return carry

        lax.fori_loop(0, ROWS_PER_SUB // ZCHUNK, zcp, None)
        plsc.subcore_barrier()

        pltpu.sync_copy(idx_hbm.at[wid], idx_v)

        def sup(j, carry):
            pltpu.sync_copy(rows_hbm.at[pl.ds(base + j * SUP_S, SUP_S)], rows_v)
            descs = [
                pltpu.async_copy(
                    rows_v.at[pl.ds(k * CHUNK, CHUNK)],
                    agg_sh.at[idx_v.at[j * SPC_S + k]],
                    sem,
                    add=True,
                )
                for k in range(SPC_S)
            ]
            for d in descs:
                d.wait()
            return carry

        lax.fori_loop(0, N_SUP_S, sup, None)
        plsc.subcore_barrier()

        def cp(k, carry):
            r = s * ROWS_PER_SUB + k * ZCHUNK
            pltpu.sync_copy(agg_sh.at[pl.ds(r, ZCHUNK)], zbuf_v)
            pltpu.sync_copy(zbuf_v, out_hbm.at[pl.ds(c * N_A_PAD + r, ZCHUNK)])
            return carry

        lax.fori_loop(0, ROWS_PER_SUB // ZCHUNK, cp, None)

    _sc_cache.update(g1=_sc_gather1, scat=_sc_scatter)
    return _sc_cache["g1"], _sc_cache["scat"]


def _pad_mask(i, blk):
    row = i * blk + lax.broadcasted_iota(jnp.int32, (blk, H), 0)
    return (row % HALF_PAD) < HALF


def _init_body(bond_ref, xg_ref, wb_ref, wx_ref, b_ref, out_ref):
    acc = jnp.dot(bond_ref[...], wb_ref[...],
                  preferred_element_type=jnp.float32, precision=_HIGHEST)
    acc += jnp.dot(xg_ref[...], wx_ref[...],
                   preferred_element_type=jnp.float32, precision=_HIGHEST)
    h0 = jnp.maximum(acc + b_ref[...], 0.0)
    out_ref[...] = jnp.where(_pad_mask(pl.program_id(0), BLK), h0, 0.0)


_tc_init = pl.pallas_call(
    _init_body,
    grid=(NBLK,),
    in_specs=[
        pl.BlockSpec((BLK, BOND_DIM), lambda i: (i, 0)),
        pl.BlockSpec((BLK, H), lambda i: (i, 0)),
        pl.BlockSpec((BOND_DIM, H), lambda i: (0, 0)),
        pl.BlockSpec((H, H), lambda i: (0, 0)),
        pl.BlockSpec((1, H), lambda i: (0, 0)),
    ],
    out_specs=pl.BlockSpec((BLK, H), lambda i: (i, 0)),
    out_shape=jax.ShapeDtypeStruct((E_PAD, H), jnp.float32),
)


def _combine_body(a_ref, b_ref, out_ref):
    out_ref[...] = a_ref[...] + b_ref[...]


_tc_combine = pl.pallas_call(
    _combine_body,
    grid=(N_A_PAD // BLK,),
    in_specs=[
        pl.BlockSpec((BLK, H), lambda i: (i, 0)),
        pl.BlockSpec((BLK, H), lambda i: (i, 0)),
    ],
    out_specs=pl.BlockSpec((BLK, H), lambda i: (i, 0)),
    out_shape=jax.ShapeDtypeStruct((N_A_PAD, H), jnp.float32),
)


def _depth_body(h0_ref, g_ref, hrev_ref, w_ref, b_ref, out_ref):
    m = g_ref[...] - hrev_ref[...]
    acc = jnp.dot(m, w_ref[...],
                  preferred_element_type=jnp.float32, precision=_HIGHEST)
    h = jnp.maximum(h0_ref[...] + acc + b_ref[...], 0.0)
    out_ref[...] = jnp.where(_pad_mask(pl.program_id(0), BLK), h, 0.0)


_tc_depth = pl.pallas_call(
    _depth_body,
    grid=(NBLK,),
    in_specs=[
        pl.BlockSpec((BLK, H), lambda i: (i, 0)),
        pl.BlockSpec((BLK, H), lambda i: (i, 0)),
        pl.BlockSpec((BLK, H), lambda i: ((i + NBLK // 2) % NBLK, 0)),
        pl.BlockSpec((H, H), lambda i: (0, 0)),
        pl.BlockSpec((1, H), lambda i: (0, 0)),
    ],
    out_specs=pl.BlockSpec((BLK, H), lambda i: (i, 0)),
    out_shape=jax.ShapeDtypeStruct((E_PAD, H), jnp.float32),
)


def _final_body(atom_ref, agga_ref, aggb_ref, a2m_ref, molf_ref,
                wx_ref, wm_ref, b_ref, out_ref):
    i = pl.program_id(0)
    mv = agga_ref[...] + aggb_ref[...]
    hv = jnp.dot(atom_ref[...], wx_ref[...],
                 preferred_element_type=jnp.float32, precision=_HIGHEST)
    hv += jnp.dot(mv, wm_ref[...],
                  preferred_element_type=jnp.float32, precision=_HIGHEST)
    hv = jnp.maximum(hv + b_ref[...], 0.0)
    seg = a2m_ref[0, 0, :]
    onehot = (lax.broadcasted_iota(jnp.int32, (N_MOL, BLKA), 0)
              == seg[None, :]).astype(jnp.float32)
    contrib = jnp.dot(onehot, hv,
                      preferred_element_type=jnp.float32, precision=_HIGHEST)

    @pl.when(i == 0)
    def _():
        out_ref[:, :H] = contrib
        out_ref[:, H:] = molf_ref[...]

    @pl.when(i != 0)
    def _():
        out_ref[:, :H] = out_ref[:, :H] + contrib


_tc_final = pl.pallas_call(
    _final_body,
    grid=(NBLKA,),
    in_specs=[
        pl.BlockSpec((BLKA, H), lambda i: (i, 0)),
        pl.BlockSpec((BLKA, H), lambda i: (i, 0)),
        pl.BlockSpec((BLKA, H), lambda i: (i, 0)),
        pl.BlockSpec((1, 1, BLKA), lambda i: (i, 0, 0)),
        pl.BlockSpec((N_MOL, H), lambda i: (0, 0)),
        pl.BlockSpec((H, H), lambda i: (0, 0)),
        pl.BlockSpec((H, H), lambda i: (0, 0)),
        pl.BlockSpec((1, H), lambda i: (0, 0)),
    ],
    out_specs=pl.BlockSpec((N_MOL, 2 * H), lambda i: (0, 0)),
    out_shape=jax.ShapeDtypeStruct((N_MOL, 2 * H), jnp.float32),
)


def _mid_pad(x, fill):
    z = jnp.full((PAD,) + x.shape[1:], fill, dtype=x.dtype)
    return jnp.concatenate([x[:HALF], z, x[HALF:], z], axis=0)


def kernel(atom_features, bond_features, molecule_features, Wi_w, Wi_b,
           Wm_w, Wm_b, Wa_w, Wa_b, bond_index, atom_to_molecule, b2rev):
    src = bond_index[0].astype(jnp.int32)
    dst = bond_index[1].astype(jnp.int32)

    src_p = _mid_pad(src, 0)
    dst_p = _mid_pad(dst, 0)
    bond_p = _mid_pad(bond_features, 0.0)

    src_t = src_p.reshape(NW, N_CHUNKS, CHUNK)
    dst_t = dst_p.reshape(NW, N_CHUNKS, CHUNK)

    wi_t = Wi_w.T
    wb_t = wi_t[:BOND_DIM]
    wx_t = wi_t[BOND_DIM:]
    wm_t = Wm_w.T
    wa_t = Wa_w.T
    wax_t = wa_t[:H]
    wam_t = wa_t[H:]

    wi_b = Wi_b.reshape(1, H)
    wm_b = Wm_b.reshape(1, H)
    wa_b = Wa_b.reshape(1, H)

    a2m_t = atom_to_molecule.astype(jnp.int32).reshape(NBLKA, 1, BLKA)

    sc_gather1, sc_scatter = _sc_kernels()

    xg = sc_gather1(atom_features, src_t)
    h0 = _tc_init(bond_p, xg, wb_t, wx_t, wi_b)

    h = h0
    for _ in range(DEPTH):
        agg2 = sc_scatter(h, dst_t)
        agg = _tc_combine(agg2[:N_A_PAD], agg2[N_A_PAD:])
        g = sc_gather1(agg, src_t)
        h = _tc_depth(h0, g, h, wm_t, wm_b)

    agg2f = sc_scatter(h, src_t)
    return _tc_final(atom_features, agg2f[:N_ATOMS], agg2f[N_A_PAD:N_A_PAD + N_ATOMS],
                     a2m_t, molecule_features, wax_t, wam_t, wa_b)


# speedup vs baseline: 2.3242x; 1.3879x over previous
import functools

import jax
import jax.numpy as jnp
from jax import lax
from jax.experimental import pallas as pl
from jax.experimental.pallas import tpu as pltpu
from jax.experimental.pallas import tpu_sc as plsc

N_ATOMS = 10000
N_MOL = 500
H = 128
BOND_DIM = 16
DEPTH = 3

HALF = 80000
HALF_PAD = 81920
E_PAD = 2 * HALF_PAD
PAD = HALF_PAD - HALF

NC, NS = 2, 16
NW = NC * NS
PER_W = E_PAD // NW
CHUNK = 128
N_CHUNKS = PER_W // CHUNK
SUP = 512
SPC = SUP // CHUNK
N_SUP = PER_W // SUP
SUP_S = 256
SPC_S = SUP_S // CHUNK
N_SUP_S = PER_W // SUP_S

N_A_PAD = 10240
ROWS_PER_SUB = N_A_PAD // NS
ZCHUNK = 64

BLK = 1024
NBLK = E_PAD // BLK
BLKA = 1000
NBLKA = N_ATOMS // BLKA

_HIGHEST = lax.Precision.HIGHEST


def _worker_id():
    return lax.axis_index("c") * NS + lax.axis_index("s")


_sc_cache = {}


def _sc_kernels():
    if _sc_cache:
        return _sc_cache["g1"], _sc_cache["scat"]

    mesh = plsc.VectorSubcoreMesh(core_axis_name="c", subcore_axis_name="s",
                                  num_cores=NC, num_subcores=NS)

    @functools.partial(
        pl.kernel,
        out_type=jax.ShapeDtypeStruct((E_PAD, H), jnp.float32),
        mesh=mesh,
        scratch_types=[
            pltpu.VMEM((N_CHUNKS, CHUNK), jnp.int32),
            pltpu.VMEM((SUP, H), jnp.float32),
            pltpu.SemaphoreType.DMA,
        ],
    )
    def _sc_gather1(tab_hbm, idx_hbm, out_hbm, idx_v, rows_v, sem):
        wid = _worker_id()
        base = wid * PER_W
        pltpu.sync_copy(idx_hbm.at[wid], idx_v)

        def sup(j, carry):
            descs = [
                pltpu.async_copy(
                    tab_hbm.at[idx_v.at[j * SPC + k]],
                    rows_v.at[pl.ds(k * CHUNK, CHUNK)],
                    sem,
                )
                for k in range(SPC)
            ]
            for d in descs:
                d.wait()
            pltpu.sync_copy(rows_v, out_hbm.at[pl.ds(base + j * SUP, SUP)])
            return carry

        lax.fori_loop(0, N_SUP, sup, None)

    @functools.partial(
        pl.kernel,
        out_type=jax.ShapeDtypeStruct((2 * N_A_PAD, H), jnp.float32),
        mesh=mesh,
        scratch_types=[
            pltpu.VMEM((N_CHUNKS, CHUNK), jnp.int32),
            pltpu.VMEM((SUP_S, H), jnp.float32),
            pltpu.VMEM((ZCHUNK, H), jnp.float32),
            pltpu.VMEM_SHARED((N_A_PAD, H), jnp.float32),
            pltpu.SemaphoreType.DMA,
        ],
    )
    def _sc_scatter(rows_hbm, idx_hbm, out_hbm, idx_v, rows_v, zbuf_v,
                    agg_sh, sem):
        c = lax.axis_index("c")
        s = lax.axis_index("s")
        wid = c * NS + s
        base = wid * PER_W

        def zrow(i, carry):
            for cc in range(H // 16):
                zbuf_v[i, pl.ds(cc * 16, 16)] = jnp.zeros((16,), jnp.float32)
            return carry

        lax.fori_loop(0, ZCHUNK, zrow, None)

        def zcp(k, carry):
            r = s * ROWS_PER_SUB + k * ZCHUNK
            pltpu.sync_copy(zbuf_v, agg_sh.at[pl.ds(r, ZCHUNK)])
            return carry

        lax.fori_loop(0, ROWS_PER_SUB // ZCHUNK, zcp, None)
        plsc.subcore_barrier()

        pltpu.sync_copy(idx_hbm.at[wid], idx_v)

        def sup(j, carry):
            pltpu.sync_copy(rows_hbm.at[pl.ds(base + j * SUP_S, SUP_S)], rows_v)
            descs = [
                pltpu.async_copy(
                    rows_v.at[pl.ds(k * CHUNK, CHUNK)],
                    agg_sh.at[idx_v.at[j * SPC_S + k]],
                    sem,
                    add=True,
                )
                for k in range(SPC_S)
            ]
            for d in descs:
                d.wait()
            return carry

        lax.fori_loop(0, N_SUP_S, sup, None)
        plsc.subcore_barrier()

        def cp(k, carry):
            r = s * ROWS_PER_SUB + k * ZCHUNK
            pltpu.sync_copy(agg_sh.at[pl.ds(r, ZCHUNK)], zbuf_v)
            pltpu.sync_copy(zbuf_v, out_hbm.at[pl.ds(c * N_A_PAD + r, ZCHUNK)])
            return carry

        lax.fori_loop(0, ROWS_PER_SUB // ZCHUNK, cp, None)

    _sc_cache.update(g1=_sc_gather1, scat=_sc_scatter)
    return _sc_cache["g1"], _sc_cache["scat"]


def _pad_mask(i, blk):
    row = i * blk + lax.broadcasted_iota(jnp.int32, (blk, H), 0)
    return (row % HALF_PAD) < HALF


def _init_body(bond_ref, xg_ref, wb_ref, wx_ref, b_ref, out_ref):
    acc = jnp.dot(bond_ref[...], wb_ref[...],
                  preferred_element_type=jnp.float32, precision=_HIGHEST)
    acc += jnp.dot(xg_ref[...], wx_ref[...],
                   preferred_element_type=jnp.float32, precision=_HIGHEST)
    h0 = jnp.maximum(acc + b_ref[...], 0.0)
    out_ref[...] = jnp.where(_pad_mask(pl.program_id(0), BLK), h0, 0.0)


_tc_init = pl.pallas_call(
    _init_body,
    grid=(NBLK,),
    in_specs=[
        pl.BlockSpec((BLK, BOND_DIM), lambda i: (i, 0)),
        pl.BlockSpec((BLK, H), lambda i: (i, 0)),
        pl.BlockSpec((BOND_DIM, H), lambda i: (0, 0)),
        pl.BlockSpec((H, H), lambda i: (0, 0)),
        pl.BlockSpec((1, H), lambda i: (0, 0)),
    ],
    out_specs=pl.BlockSpec((BLK, H), lambda i: (i, 0)),
    out_shape=jax.ShapeDtypeStruct((E_PAD, H), jnp.float32),
)


def _combine_body(a_ref, b_ref, out_ref):
    out_ref[...] = a_ref[...] + b_ref[...]


_tc_combine = pl.pallas_call(
    _combine_body,
    grid=(N_A_PAD // BLK,),
    in_specs=[
        pl.BlockSpec((BLK, H), lambda i: (i, 0)),
        pl.BlockSpec((BLK, H), lambda i: (i, 0)),
    ],
    out_specs=pl.BlockSpec((BLK, H), lambda i: (i, 0)),
    out_shape=jax.ShapeDtypeStruct((N_A_PAD, H), jnp.float32),
)


def _depth_body(h0_ref, g_ref, hrev_ref, w_ref, b_ref, out_ref):
    m = g_ref[...] - hrev_ref[...]
    acc = jnp.dot(m, w_ref[...],
                  preferred_element_type=jnp.float32, precision=_HIGHEST)
    h = jnp.maximum(h0_ref[...] + acc + b_ref[...], 0.0)
    out_ref[...] = jnp.where(_pad_mask(pl.program_id(0), BLK), h, 0.0)


_tc_depth = pl.pallas_call(
    _depth_body,
    grid=(NBLK,),
    in_specs=[
        pl.BlockSpec((BLK, H), lambda i: (i, 0)),
        pl.BlockSpec((BLK, H), lambda i: (i, 0)),
        pl.BlockSpec((BLK, H), lambda i: ((i + NBLK // 2) % NBLK, 0)),
        pl.BlockSpec((H, H), lambda i: (0, 0)),
        pl.BlockSpec((1, H), lambda i: (0, 0)),
    ],
    out_specs=pl.BlockSpec((BLK, H), lambda i: (i, 0)),
    out_shape=jax.ShapeDtypeStruct((E_PAD, H), jnp.float32),
)


def _final_body(atom_ref, agga_ref, aggb_ref, a2m_ref, molf_ref,
                wx_ref, wm_ref, b_ref, out_ref):
    i = pl.program_id(0)
    mv = agga_ref[...] + aggb_ref[...]
    hv = jnp.dot(atom_ref[...], wx_ref[...],
                 preferred_element_type=jnp.float32, precision=_HIGHEST)
    hv += jnp.dot(mv, wm_ref[...],
                  preferred_element_type=jnp.float32, precision=_HIGHEST)
    hv = jnp.maximum(hv + b_ref[...], 0.0)
    seg = a2m_ref[0, 0, :]
    onehot = (lax.broadcasted_iota(jnp.int32, (N_MOL, BLKA), 0)
              == seg[None, :]).astype(jnp.float32)
    contrib = jnp.dot(onehot, hv,
                      preferred_element_type=jnp.float32, precision=_HIGHEST)

    @pl.when(i == 0)
    def _():
        out_ref[:, :H] = contrib
        out_ref[:, H:] = molf_ref[...]

    @pl.when(i != 0)
    def _():
        out_ref[:, :H] = out_ref[:, :H] + contrib


_tc_final = pl.pallas_call(
    _final_body,
    grid=(NBLKA,),
    in_specs=[
        pl.BlockSpec((BLKA, H), lambda i: (i, 0)),
        pl.BlockSpec((BLKA, H), lambda i: (i, 0)),
        pl.BlockSpec((BLKA, H), lambda i: (i, 0)),
        pl.BlockSpec((1, 1, BLKA), lambda i: (i, 0, 0)),
        pl.BlockSpec((N_MOL, H), lambda i: (0, 0)),
        pl.BlockSpec((H, H), lambda i: (0, 0)),
        pl.BlockSpec((H, H), lambda i: (0, 0)),
        pl.BlockSpec((1, H), lambda i: (0, 0)),
    ],
    out_specs=pl.BlockSpec((N_MOL, 2 * H), lambda i: (0, 0)),
    out_shape=jax.ShapeDtypeStruct((N_MOL, 2 * H), jnp.float32),
)


def _mid_pad(x):
    z = jnp.zeros((PAD,) + x.shape[1:], dtype=x.dtype)
    return jnp.concatenate([x[:HALF], z, x[HALF:], z], axis=0)


def _mid_pad_idx(x):
    z = jnp.arange(PAD, dtype=x.dtype) % N_ATOMS
    return jnp.concatenate([x[:HALF], z, x[HALF:], z], axis=0)


def kernel(atom_features, bond_features, molecule_features, Wi_w, Wi_b,
           Wm_w, Wm_b, Wa_w, Wa_b, bond_index, atom_to_molecule, b2rev):
    src = bond_index[0].astype(jnp.int32)
    dst = bond_index[1].astype(jnp.int32)

    src_p = _mid_pad_idx(src)
    dst_p = _mid_pad_idx(dst)
    bond_p = _mid_pad(bond_features)

    src_t = src_p.reshape(NW, N_CHUNKS, CHUNK)
    dst_t = dst_p.reshape(NW, N_CHUNKS, CHUNK)

    wi_t = Wi_w.T
    wb_t = wi_t[:BOND_DIM]
    wx_t = wi_t[BOND_DIM:]
    wm_t = Wm_w.T
    wa_t = Wa_w.T
    wax_t = wa_t[:H]
    wam_t = wa_t[H:]

    wi_b = Wi_b.reshape(1, H)
    wm_b = Wm_b.reshape(1, H)
    wa_b = Wa_b.reshape(1, H)

    a2m_t = atom_to_molecule.astype(jnp.int32).reshape(NBLKA, 1, BLKA)

    sc_gather1, sc_scatter = _sc_kernels()

    xg = sc_gather1(atom_features, src_t)
    h0 = _tc_init(bond_p, xg, wb_t, wx_t, wi_b)

    h = h0
    for _ in range(DEPTH):
        agg2 = sc_scatter(h, dst_t)
        agg = _tc_combine(agg2[:N_A_PAD], agg2[N_A_PAD:])
        g = sc_gather1(agg, src_t)
        h = _tc_depth(h0, g, h, wm_t, wm_b)

    agg2f = sc_scatter(h, src_t)
    return _tc_final(atom_features, agg2f[:N_ATOMS], agg2f[N_A_PAD:N_A_PAD + N_ATOMS],
                     a2m_t, molecule_features, wax_t, wam_t, wa_b)


# speedup vs baseline: 2.6577x; 1.1435x over previous
import functools

import jax
import jax.numpy as jnp
from jax import lax
from jax.experimental import pallas as pl
from jax.experimental.pallas import tpu as pltpu
from jax.experimental.pallas import tpu_sc as plsc

N_ATOMS = 10000
N_MOL = 500
H = 128
BOND_DIM = 16
DEPTH = 3

HALF = 80000
HALF_PAD = 81920
E_PAD = 2 * HALF_PAD
PAD = HALF_PAD - HALF

NC, NS = 2, 16
NW = NC * NS
PER_W = E_PAD // NW
CHUNK = 128
N_CHUNKS = PER_W // CHUNK
SUP = 256
SPC = SUP // CHUNK
N_SUP = PER_W // SUP
SUP_S = CHUNK
N_SUP_S = PER_W // SUP_S

N_A_PAD = 10240
ROWS_PER_SUB = N_A_PAD // NS
ZCHUNK = 64

BLK = 2048
NBLK = E_PAD // BLK
BLKA = 1000
NBLKA = N_ATOMS // BLKA

_HIGHEST = lax.Precision.HIGHEST


def _worker_id():
    return lax.axis_index("c") * NS + lax.axis_index("s")


_sc_cache = {}


def _sc_kernels():
    if _sc_cache:
        return _sc_cache["g1"], _sc_cache["scat"]

    mesh = plsc.VectorSubcoreMesh(core_axis_name="c", subcore_axis_name="s",
                                  num_cores=NC, num_subcores=NS)

    @functools.partial(
        pl.kernel,
        out_type=jax.ShapeDtypeStruct((E_PAD, H), jnp.float32),
        mesh=mesh,
        scratch_types=[
            pltpu.VMEM((N_CHUNKS, CHUNK), jnp.int32),
            pltpu.VMEM((SUP, H), jnp.float32),
            pltpu.VMEM((SUP, H), jnp.float32),
            pltpu.SemaphoreType.DMA,
            pltpu.SemaphoreType.DMA,
            pltpu.SemaphoreType.DMA,
            pltpu.SemaphoreType.DMA,
        ],
    )
    def _sc_gather1(tab_hbm, idx_hbm, out_hbm, idx_v, b0, b1,
                    gs0, gs1, os0, os1):
        wid = lax.axis_index("c") * NS + lax.axis_index("s")
        base = wid * PER_W
        pltpu.sync_copy(idx_hbm.at[wid], idx_v)

        def fire_g(j, buf, sem):
            for k in range(SPC):
                pltpu.async_copy(tab_hbm.at[idx_v.at[j * SPC + k]],
                                 buf.at[pl.ds(k * CHUNK, CHUNK)], sem)

        def drain_g(buf, sem):
            for k in range(SPC):
                pltpu.make_async_copy(tab_hbm.at[idx_v.at[k]],
                                      buf.at[pl.ds(k * CHUNK, CHUNK)],
                                      sem).wait()

        def fire_out(j, buf, sem):
            pltpu.async_copy(buf, out_hbm.at[pl.ds(base + j * SUP, SUP)], sem)

        def wait_out(j, buf, sem):
            pltpu.make_async_copy(buf, out_hbm.at[pl.ds(base + j * SUP, SUP)],
                                  sem).wait()

        fire_g(0, b0, gs0)

        def body(jj, carry):
            j0 = 2 * jj
            drain_g(b0, gs0)

            @pl.when(jj > 0)
            def _():
                wait_out(j0 - 1, b1, os1)

            fire_g(j0 + 1, b1, gs1)
            fire_out(j0, b0, os0)
            drain_g(b1, gs1)
            wait_out(j0, b0, os0)

            @pl.when(jj < N_SUP // 2 - 1)
            def _():
                fire_g(j0 + 2, b0, gs0)

            fire_out(j0 + 1, b1, os1)
            return carry

        lax.fori_loop(0, N_SUP // 2, body, None)
        wait_out(N_SUP - 1, b1, os1)

    @functools.partial(
        pl.kernel,
        out_type=jax.ShapeDtypeStruct((2 * N_A_PAD, H), jnp.float32),
        mesh=mesh,
        scratch_types=[
            pltpu.VMEM((N_CHUNKS, CHUNK), jnp.int32),
            pltpu.VMEM((SUP_S, H), jnp.float32),
            pltpu.VMEM((SUP_S, H), jnp.float32),
            pltpu.VMEM((ZCHUNK, H), jnp.float32),
            pltpu.VMEM_SHARED((N_A_PAD, H), jnp.float32),
            pltpu.SemaphoreType.DMA,
            pltpu.SemaphoreType.DMA,
            pltpu.SemaphoreType.DMA,
            pltpu.SemaphoreType.DMA,
        ],
    )
    def _sc_scatter(rows_hbm, idx_hbm, out_hbm, idx_v, b0, b1, zbuf_v,
                    agg_sh, ls0, ls1, ss0, ss1):
        c = lax.axis_index("c")
        s = lax.axis_index("s")
        wid = c * NS + s
        base = wid * PER_W

        def zrow(i, carry):
            for cc in range(H // 16):
                zbuf_v[i, pl.ds(cc * 16, 16)] = jnp.zeros((16,), jnp.float32)
            return carry

        lax.fori_loop(0, ZCHUNK, zrow, None)

        def zcp(k, carry):
            r = s * ROWS_PER_SUB + k * ZCHUNK
            pltpu.sync_copy(zbuf_v, agg_sh.at[pl.ds(r, ZCHUNK)])
            return carry

        lax.fori_loop(0, ROWS_PER_SUB // ZCHUNK, zcp, None)
        plsc.subcore_barrier()

        pltpu.sync_copy(idx_hbm.at[wid], idx_v)

        def fire_load(j, buf, sem):
            pltpu.async_copy(rows_hbm.at[pl.ds(base + j * SUP_S, SUP_S)],
                             buf, sem)

        def wait_load(buf, sem):
            pltpu.make_async_copy(rows_hbm.at[pl.ds(base, SUP_S)], buf,
                                  sem).wait()

        def fire_add(j, buf, sem):
            pltpu.async_copy(buf, agg_sh.at[idx_v.at[j]], sem, add=True)

        def drain_add(buf, sem):
            pltpu.make_async_copy(buf, agg_sh.at[idx_v.at[0]], sem).wait()

        fire_load(0, b0, ls0)

        def sup(jj, carry):
            j0 = 2 * jj
            wait_load(b0, ls0)
            fire_add(j0, b0, ss0)

            @pl.when(jj > 0)
            def _():
                drain_add(b1, ss1)

            fire_load(j0 + 1, b1, ls1)
            drain_add(b0, ss0)

            @pl.when(jj < N_SUP_S // 2 - 1)
            def _():
                fire_load(j0 + 2, b0, ls0)

            wait_load(b1, ls1)
            fire_add(j0 + 1, b1, ss1)
            return carry

        lax.fori_loop(0, N_SUP_S // 2, sup, None)
        drain_add(b1, ss1)
        plsc.subcore_barrier()

        def cp(k, carry):
            r = s * ROWS_PER_SUB + k * ZCHUNK
            pltpu.sync_copy(agg_sh.at[pl.ds(r, ZCHUNK)], zbuf_v)
            pltpu.sync_copy(zbuf_v, out_hbm.at[pl.ds(c * N_A_PAD + r, ZCHUNK)])
            return carry

        lax.fori_loop(0, ROWS_PER_SUB // ZCHUNK, cp, None)

    _sc_cache.update(g1=_sc_gather1, scat=_sc_scatter)
    return _sc_cache["g1"], _sc_cache["scat"]


def _pad_mask(i, blk):
    row = i * blk + lax.broadcasted_iota(jnp.int32, (blk, H), 0)
    return (row % HALF_PAD) < HALF


def _init_body(bond_ref, xg_ref, wb_ref, wx_ref, b_ref, out_ref):
    acc = jnp.dot(bond_ref[...], wb_ref[...],
                  preferred_element_type=jnp.float32, precision=_HIGHEST)
    acc += jnp.dot(xg_ref[...], wx_ref[...],
                   preferred_element_type=jnp.float32, precision=_HIGHEST)
    h0 = jnp.maximum(acc + b_ref[...], 0.0)
    out_ref[...] = jnp.where(_pad_mask(pl.program_id(0), BLK), h0, 0.0)


_tc_init = pl.pallas_call(
    _init_body,
    grid=(NBLK,),
    in_specs=[
        pl.BlockSpec((BLK, BOND_DIM), lambda i: (i, 0)),
        pl.BlockSpec((BLK, H), lambda i: (i, 0)),
        pl.BlockSpec((BOND_DIM, H), lambda i: (0, 0)),
        pl.BlockSpec((H, H), lambda i: (0, 0)),
        pl.BlockSpec((1, H), lambda i: (0, 0)),
    ],
    out_specs=pl.BlockSpec((BLK, H), lambda i: (i, 0)),
    out_shape=jax.ShapeDtypeStruct((E_PAD, H), jnp.float32),
)


def _combine_body(a_ref, b_ref, out_ref):
    out_ref[...] = a_ref[...] + b_ref[...]


_tc_combine = pl.pallas_call(
    _combine_body,
    grid=(N_A_PAD // 1024,),
    in_specs=[
        pl.BlockSpec((1024, H), lambda i: (i, 0)),
        pl.BlockSpec((1024, H), lambda i: (i, 0)),
    ],
    out_specs=pl.BlockSpec((1024, H), lambda i: (i, 0)),
    out_shape=jax.ShapeDtypeStruct((N_A_PAD, H), jnp.float32),
)


def _depth_body(h0_ref, g_ref, hrev_ref, w_ref, b_ref, out_ref):
    m = g_ref[...] - hrev_ref[...]
    acc = jnp.dot(m, w_ref[...],
                  preferred_element_type=jnp.float32, precision=_HIGHEST)
    h = jnp.maximum(h0_ref[...] + acc + b_ref[...], 0.0)
    out_ref[...] = jnp.where(_pad_mask(pl.program_id(0), BLK), h, 0.0)


_tc_depth = pl.pallas_call(
    _depth_body,
    grid=(NBLK,),
    in_specs=[
        pl.BlockSpec((BLK, H), lambda i: (i, 0)),
        pl.BlockSpec((BLK, H), lambda i: (i, 0)),
        pl.BlockSpec((BLK, H), lambda i: ((i + NBLK // 2) % NBLK, 0)),
        pl.BlockSpec((H, H), lambda i: (0, 0)),
        pl.BlockSpec((1, H), lambda i: (0, 0)),
    ],
    out_specs=pl.BlockSpec((BLK, H), lambda i: (i, 0)),
    out_shape=jax.ShapeDtypeStruct((E_PAD, H), jnp.float32),
)


def _final_body(atom_ref, agga_ref, aggb_ref, a2m_ref, molf_ref,
                wx_ref, wm_ref, b_ref, out_ref):
    i = pl.program_id(0)
    mv = agga_ref[...] + aggb_ref[...]
    hv = jnp.dot(atom_ref[...], wx_ref[...],
                 preferred_element_type=jnp.float32, precision=_HIGHEST)
    hv += jnp.dot(mv, wm_ref[...],
                  preferred_element_type=jnp.float32, precision=_HIGHEST)
    hv = jnp.maximum(hv + b_ref[...], 0.0)
    seg = a2m_ref[0, 0, :]
    onehot = (lax.broadcasted_iota(jnp.int32, (N_MOL, BLKA), 0)
              == seg[None, :]).astype(jnp.float32)
    contrib = jnp.dot(onehot, hv,
                      preferred_element_type=jnp.float32, precision=_HIGHEST)

    @pl.when(i == 0)
    def _():
        out_ref[:, :H] = contrib
        out_ref[:, H:] = molf_ref[...]

    @pl.when(i != 0)
    def _():
        out_ref[:, :H] = out_ref[:, :H] + contrib


_tc_final = pl.pallas_call(
    _final_body,
    grid=(NBLKA,),
    in_specs=[
        pl.BlockSpec((BLKA, H), lambda i: (i, 0)),
        pl.BlockSpec((BLKA, H), lambda i: (i, 0)),
        pl.BlockSpec((BLKA, H), lambda i: (i, 0)),
        pl.BlockSpec((1, 1, BLKA), lambda i: (i, 0, 0)),
        pl.BlockSpec((N_MOL, H), lambda i: (0, 0)),
        pl.BlockSpec((H, H), lambda i: (0, 0)),
        pl.BlockSpec((H, H), lambda i: (0, 0)),
        pl.BlockSpec((1, H), lambda i: (0, 0)),
    ],
    out_specs=pl.BlockSpec((N_MOL, 2 * H), lambda i: (0, 0)),
    out_shape=jax.ShapeDtypeStruct((N_MOL, 2 * H), jnp.float32),
)


def _mid_pad(x):
    z = jnp.zeros((PAD,) + x.shape[1:], dtype=x.dtype)
    return jnp.concatenate([x[:HALF], z, x[HALF:], z], axis=0)


def _mid_pad_idx(x):
    z = jnp.arange(PAD, dtype=x.dtype) % N_ATOMS
    return jnp.concatenate([x[:HALF], z, x[HALF:], z], axis=0)


def kernel(atom_features, bond_features, molecule_features, Wi_w, Wi_b,
           Wm_w, Wm_b, Wa_w, Wa_b, bond_index, atom_to_molecule, b2rev):
    src = bond_index[0].astype(jnp.int32)
    dst = bond_index[1].astype(jnp.int32)

    src_p = _mid_pad_idx(src)
    dst_p = _mid_pad_idx(dst)
    bond_p = _mid_pad(bond_features)

    src_t = src_p.reshape(NW, N_CHUNKS, CHUNK)
    dst_t = dst_p.reshape(NW, N_CHUNKS, CHUNK)

    wi_t = Wi_w.T
    wb_t = wi_t[:BOND_DIM]
    wx_t = wi_t[BOND_DIM:]
    wm_t = Wm_w.T
    wa_t = Wa_w.T
    wax_t = wa_t[:H]
    wam_t = wa_t[H:]

    wi_b = Wi_b.reshape(1, H)
    wm_b = Wm_b.reshape(1, H)
    wa_b = Wa_b.reshape(1, H)

    a2m_t = atom_to_molecule.astype(jnp.int32).reshape(NBLKA, 1, BLKA)

    sc_gather1, sc_scatter = _sc_kernels()

    xg = sc_gather1(atom_features, src_t)
    h0 = _tc_init(bond_p, xg, wb_t, wx_t, wi_b)

    h = h0
    for _ in range(DEPTH):
        agg2 = sc_scatter(h, dst_t)
        agg = _tc_combine(agg2[:N_A_PAD], agg2[N_A_PAD:])
        g = sc_gather1(agg, src_t)
        h = _tc_depth(h0, g, h, wm_t, wm_b)

    agg2f = sc_scatter(h, src_t)
    return _tc_final(atom_features, agg2f[:N_ATOMS], agg2f[N_A_PAD:N_A_PAD + N_ATOMS],
                     a2m_t, molecule_features, wax_t, wam_t, wa_b)


# speedup vs baseline: 2.8239x; 1.0625x over previous
import functools

import jax
import jax.numpy as jnp
from jax import lax
from jax.experimental import pallas as pl
from jax.experimental.pallas import tpu as pltpu
from jax.experimental.pallas import tpu_sc as plsc

N_ATOMS = 10000
N_MOL = 500
H = 128
BOND_DIM = 16
DEPTH = 3

HALF = 80000
HALF_PAD = 81920
E_PAD = 2 * HALF_PAD
PAD = HALF_PAD - HALF

NC, NS = 2, 16
NW = NC * NS
PER_W = E_PAD // NW
CHUNK = 128
N_CHUNKS = PER_W // CHUNK
SUP = 256
SPC = SUP // CHUNK
N_SUP = PER_W // SUP
SUP_S = CHUNK
N_SUP_S = PER_W // SUP_S

N_A_PAD = 10240
ROWS_PER_SUB = N_A_PAD // NS
ZCHUNK = 64

BLK = 2048
NBLK = E_PAD // BLK
BLKA = 1000
NBLKA = N_ATOMS // BLKA

_HIGHEST = lax.Precision.HIGHEST


def _worker_id():
    return lax.axis_index("c") * NS + lax.axis_index("s")


_sc_cache = {}


def _sc_kernels():
    if _sc_cache:
        return _sc_cache["g1"], _sc_cache["scat"]

    mesh = plsc.VectorSubcoreMesh(core_axis_name="c", subcore_axis_name="s",
                                  num_cores=NC, num_subcores=NS)

    @functools.partial(
        pl.kernel,
        out_type=jax.ShapeDtypeStruct((E_PAD, H), jnp.float32),
        mesh=mesh,
        scratch_types=[
            pltpu.VMEM((N_CHUNKS, CHUNK), jnp.int32),
            pltpu.VMEM((CHUNK, H), jnp.float32),
            pltpu.VMEM((CHUNK, H), jnp.float32),
            pltpu.VMEM_SHARED((N_A_PAD, H), jnp.float32),
            pltpu.SemaphoreType.DMA,
            pltpu.SemaphoreType.DMA,
            pltpu.SemaphoreType.DMA,
            pltpu.SemaphoreType.DMA,
        ],
    )
    def _sc_gather1(tab_hbm, idx_hbm, out_hbm, idx_v, b0, b1, tab_sh,
                    gs0, gs1, os0, os1):
        s = lax.axis_index("s")
        wid = lax.axis_index("c") * NS + lax.axis_index("s")
        base = wid * PER_W
        pltpu.sync_copy(idx_hbm.at[wid], idx_v)

        def stage(k, carry):
            r = s * ROWS_PER_SUB + k * CHUNK
            pltpu.sync_copy(tab_hbm.at[pl.ds(r, CHUNK)], b0)
            pltpu.sync_copy(b0, tab_sh.at[pl.ds(r, CHUNK)])
            return carry

        lax.fori_loop(0, ROWS_PER_SUB // CHUNK, stage, None)
        plsc.subcore_barrier()

        def fire_g(j, buf, sem):
            pltpu.async_copy(tab_sh.at[idx_v.at[j]], buf, sem)

        def drain_g(buf, sem):
            pltpu.make_async_copy(tab_sh.at[idx_v.at[0]], buf, sem).wait()

        def fire_out(j, buf, sem):
            pltpu.async_copy(buf, out_hbm.at[pl.ds(base + j * CHUNK, CHUNK)],
                             sem)

        def wait_out(buf, sem):
            pltpu.make_async_copy(buf, out_hbm.at[pl.ds(base, CHUNK)],
                                  sem).wait()

        fire_g(0, b0, gs0)

        def body(jj, carry):
            j0 = 2 * jj
            drain_g(b0, gs0)

            @pl.when(jj > 0)
            def _():
                wait_out(b1, os1)

            fire_g(j0 + 1, b1, gs1)
            fire_out(j0, b0, os0)
            drain_g(b1, gs1)
            wait_out(b0, os0)

            @pl.when(jj < N_CHUNKS // 2 - 1)
            def _():
                fire_g(j0 + 2, b0, gs0)

            fire_out(j0 + 1, b1, os1)
            return carry

        lax.fori_loop(0, N_CHUNKS // 2, body, None)
        wait_out(b1, os1)

    @functools.partial(
        pl.kernel,
        out_type=jax.ShapeDtypeStruct((2 * N_A_PAD, H), jnp.float32),
        mesh=mesh,
        scratch_types=[
            pltpu.VMEM((N_CHUNKS, CHUNK), jnp.int32),
            pltpu.VMEM((SUP_S, H), jnp.float32),
            pltpu.VMEM((SUP_S, H), jnp.float32),
            pltpu.VMEM((ZCHUNK, H), jnp.float32),
            pltpu.VMEM_SHARED((N_A_PAD, H), jnp.float32),
            pltpu.SemaphoreType.DMA,
            pltpu.SemaphoreType.DMA,
            pltpu.SemaphoreType.DMA,
            pltpu.SemaphoreType.DMA,
        ],
    )
    def _sc_scatter(rows_hbm, idx_hbm, out_hbm, idx_v, b0, b1, zbuf_v,
                    agg_sh, ls0, ls1, ss0, ss1):
        c = lax.axis_index("c")
        s = lax.axis_index("s")
        wid = c * NS + s
        base = wid * PER_W

        def zrow(i, carry):
            for cc in range(H // 16):
                zbuf_v[i, pl.ds(cc * 16, 16)] = jnp.zeros((16,), jnp.float32)
            return carry

        lax.fori_loop(0, ZCHUNK, zrow, None)

        def zcp(k, carry):
            r = s * ROWS_PER_SUB + k * ZCHUNK
            pltpu.sync_copy(zbuf_v, agg_sh.at[pl.ds(r, ZCHUNK)])
            return carry

        lax.fori_loop(0, ROWS_PER_SUB // ZCHUNK, zcp, None)
        plsc.subcore_barrier()

        pltpu.sync_copy(idx_hbm.at[wid], idx_v)

        def fire_load(j, buf, sem):
            pltpu.async_copy(rows_hbm.at[pl.ds(base + j * SUP_S, SUP_S)],
                             buf, sem)

        def wait_load(buf, sem):
            pltpu.make_async_copy(rows_hbm.at[pl.ds(base, SUP_S)], buf,
                                  sem).wait()

        def fire_add(j, buf, sem):
            pltpu.async_copy(buf, agg_sh.at[idx_v.at[j]], sem, add=True)

        def drain_add(buf, sem):
            pltpu.make_async_copy(buf, agg_sh.at[idx_v.at[0]], sem).wait()

        fire_load(0, b0, ls0)

        def sup(jj, carry):
            j0 = 2 * jj
            wait_load(b0, ls0)
            fire_add(j0, b0, ss0)

            @pl.when(jj > 0)
            def _():
                drain_add(b1, ss1)

            fire_load(j0 + 1, b1, ls1)
            drain_add(b0, ss0)

            @pl.when(jj < N_SUP_S // 2 - 1)
            def _():
                fire_load(j0 + 2, b0, ls0)

            wait_load(b1, ls1)
            fire_add(j0 + 1, b1, ss1)
            return carry

        lax.fori_loop(0, N_SUP_S // 2, sup, None)
        drain_add(b1, ss1)
        plsc.subcore_barrier()

        def cp(k, carry):
            r = s * ROWS_PER_SUB + k * ZCHUNK
            pltpu.sync_copy(agg_sh.at[pl.ds(r, ZCHUNK)], zbuf_v)
            pltpu.sync_copy(zbuf_v, out_hbm.at[pl.ds(c * N_A_PAD + r, ZCHUNK)])
            return carry

        lax.fori_loop(0, ROWS_PER_SUB // ZCHUNK, cp, None)

    _sc_cache.update(g1=_sc_gather1, scat=_sc_scatter)
    return _sc_cache["g1"], _sc_cache["scat"]


def _pad_mask(i, blk):
    row = i * blk + lax.broadcasted_iota(jnp.int32, (blk, H), 0)
    return (row % HALF_PAD) < HALF


def _init_body(bond_ref, xg_ref, wb_ref, wx_ref, b_ref, out_ref):
    acc = jnp.dot(bond_ref[...], wb_ref[...],
                  preferred_element_type=jnp.float32, precision=_HIGHEST)
    acc += jnp.dot(xg_ref[...], wx_ref[...],
                   preferred_element_type=jnp.float32, precision=_HIGHEST)
    h0 = jnp.maximum(acc + b_ref[...], 0.0)
    out_ref[...] = jnp.where(_pad_mask(pl.program_id(0), BLK), h0, 0.0)


_tc_init = pl.pallas_call(
    _init_body,
    grid=(NBLK,),
    in_specs=[
        pl.BlockSpec((BLK, BOND_DIM), lambda i: (i, 0)),
        pl.BlockSpec((BLK, H), lambda i: (i, 0)),
        pl.BlockSpec((BOND_DIM, H), lambda i: (0, 0)),
        pl.BlockSpec((H, H), lambda i: (0, 0)),
        pl.BlockSpec((1, H), lambda i: (0, 0)),
    ],
    out_specs=pl.BlockSpec((BLK, H), lambda i: (i, 0)),
    out_shape=jax.ShapeDtypeStruct((E_PAD, H), jnp.float32),
)


def _combine_body(a_ref, b_ref, out_ref):
    out_ref[...] = a_ref[...] + b_ref[...]


_tc_combine = pl.pallas_call(
    _combine_body,
    grid=(N_A_PAD // 1024,),
    in_specs=[
        pl.BlockSpec((1024, H), lambda i: (i, 0)),
        pl.BlockSpec((1024, H), lambda i: (i, 0)),
    ],
    out_specs=pl.BlockSpec((1024, H), lambda i: (i, 0)),
    out_shape=jax.ShapeDtypeStruct((N_A_PAD, H), jnp.float32),
)


def _depth_body(h0_ref, g_ref, hrev_ref, w_ref, b_ref, out_ref):
    m = g_ref[...] - hrev_ref[...]
    acc = jnp.dot(m, w_ref[...],
                  preferred_element_type=jnp.float32, precision=_HIGHEST)
    h = jnp.maximum(h0_ref[...] + acc + b_ref[...], 0.0)
    out_ref[...] = jnp.where(_pad_mask(pl.program_id(0), BLK), h, 0.0)


_tc_depth = pl.pallas_call(
    _depth_body,
    grid=(NBLK,),
    in_specs=[
        pl.BlockSpec((BLK, H), lambda i: (i, 0)),
        pl.BlockSpec((BLK, H), lambda i: (i, 0)),
        pl.BlockSpec((BLK, H), lambda i: ((i + NBLK // 2) % NBLK, 0)),
        pl.BlockSpec((H, H), lambda i: (0, 0)),
        pl.BlockSpec((1, H), lambda i: (0, 0)),
    ],
    out_specs=pl.BlockSpec((BLK, H), lambda i: (i, 0)),
    out_shape=jax.ShapeDtypeStruct((E_PAD, H), jnp.float32),
)


def _final_body(atom_ref, agga_ref, aggb_ref, a2m_ref, molf_ref,
                wx_ref, wm_ref, b_ref, out_ref):
    i = pl.program_id(0)
    mv = agga_ref[...] + aggb_ref[...]
    hv = jnp.dot(atom_ref[...], wx_ref[...],
                 preferred_element_type=jnp.float32, precision=_HIGHEST)
    hv += jnp.dot(mv, wm_ref[...],
                  preferred_element_type=jnp.float32, precision=_HIGHEST)
    hv = jnp.maximum(hv + b_ref[...], 0.0)
    seg = a2m_ref[0, 0, :]
    onehot = (lax.broadcasted_iota(jnp.int32, (N_MOL, BLKA), 0)
              == seg[None, :]).astype(jnp.float32)
    contrib = jnp.dot(onehot, hv,
                      preferred_element_type=jnp.float32, precision=_HIGHEST)

    @pl.when(i == 0)
    def _():
        out_ref[:, :H] = contrib
        out_ref[:, H:] = molf_ref[...]

    @pl.when(i != 0)
    def _():
        out_ref[:, :H] = out_ref[:, :H] + contrib


_tc_final = pl.pallas_call(
    _final_body,
    grid=(NBLKA,),
    in_specs=[
        pl.BlockSpec((BLKA, H), lambda i: (i, 0)),
        pl.BlockSpec((BLKA, H), lambda i: (i, 0)),
        pl.BlockSpec((BLKA, H), lambda i: (i, 0)),
        pl.BlockSpec((1, 1, BLKA), lambda i: (i, 0, 0)),
        pl.BlockSpec((N_MOL, H), lambda i: (0, 0)),
        pl.BlockSpec((H, H), lambda i: (0, 0)),
        pl.BlockSpec((H, H), lambda i: (0, 0)),
        pl.BlockSpec((1, H), lambda i: (0, 0)),
    ],
    out_specs=pl.BlockSpec((N_MOL, 2 * H), lambda i: (0, 0)),
    out_shape=jax.ShapeDtypeStruct((N_MOL, 2 * H), jnp.float32),
)


def _mid_pad(x):
    z = jnp.zeros((PAD,) + x.shape[1:], dtype=x.dtype)
    return jnp.concatenate([x[:HALF], z, x[HALF:], z], axis=0)


def _mid_pad_idx(x):
    z = jnp.arange(PAD, dtype=x.dtype) % N_ATOMS
    return jnp.concatenate([x[:HALF], z, x[HALF:], z], axis=0)


def kernel(atom_features, bond_features, molecule_features, Wi_w, Wi_b,
           Wm_w, Wm_b, Wa_w, Wa_b, bond_index, atom_to_molecule, b2rev):
    src = bond_index[0].astype(jnp.int32)
    dst = bond_index[1].astype(jnp.int32)

    src_p = _mid_pad_idx(src)
    dst_p = _mid_pad_idx(dst)
    bond_p = _mid_pad(bond_features)

    src_t = src_p.reshape(NW, N_CHUNKS, CHUNK)
    dst_t = dst_p.reshape(NW, N_CHUNKS, CHUNK)

    wi_t = Wi_w.T
    wb_t = wi_t[:BOND_DIM]
    wx_t = wi_t[BOND_DIM:]
    wm_t = Wm_w.T
    wa_t = Wa_w.T
    wax_t = wa_t[:H]
    wam_t = wa_t[H:]

    wi_b = Wi_b.reshape(1, H)
    wm_b = Wm_b.reshape(1, H)
    wa_b = Wa_b.reshape(1, H)

    a2m_t = atom_to_molecule.astype(jnp.int32).reshape(NBLKA, 1, BLKA)

    sc_gather1, sc_scatter = _sc_kernels()

    xg = sc_gather1(atom_features, src_t)
    h0 = _tc_init(bond_p, xg, wb_t, wx_t, wi_b)

    h = h0
    for _ in range(DEPTH):
        agg2 = sc_scatter(h, dst_t)
        agg = _tc_combine(agg2[:N_A_PAD], agg2[N_A_PAD:])
        g = sc_gather1(agg, src_t)
        h = _tc_depth(h0, g, h, wm_t, wm_b)

    agg2f = sc_scatter(h, src_t)
    return _tc_final(atom_features, agg2f[:N_ATOMS], agg2f[N_A_PAD:N_A_PAD + N_ATOMS],
                     a2m_t, molecule_features, wax_t, wam_t, wa_b)
